# baseline (speedup 1.0000x reference)
import functools
import math

import jax
import jax.numpy as jnp
from jax import lax
from jax.experimental import pallas as pl
from jax.experimental.pallas import tpu as pltpu
from jax.experimental.pallas import tpu_sc as plsc

D_MODEL = 64
SCALE = math.sqrt(D_MODEL)

NC = 2
NS = 16
NW = NC * NS
L = 16
CHUNK = 128


def _make_sc_kernel(B, D):
    assert B % (NW * CHUNK) == 0
    n_chunks = B // (NW * CHUNK)
    per_w = n_chunks * CHUNK
    mesh = plsc.VectorSubcoreMesh(core_axis_name="c", subcore_axis_name="s")

    @functools.partial(
        pl.kernel,
        mesh=mesh,
        out_type=jax.ShapeDtypeStruct((B, D), jnp.float32),
        scratch_types=[
            pltpu.VMEM((n_chunks, CHUNK), jnp.int32),
            pltpu.VMEM((CHUNK, D), jnp.float32),
            pltpu.SemaphoreType.DMA,
        ],
        compiler_params=pltpu.CompilerParams(use_tc_tiling_on_sc=False),
    )
    def k(table_hbm, idx_hbm, out_hbm, idx_v, rows, sem):
        wid = lax.axis_index("s") * NC + lax.axis_index("c")
        pltpu.sync_copy(idx_hbm.at[wid], idx_v)
        base = wid * per_w

        @pl.loop(0, n_chunks)
        def chunk_loop(j):
            pltpu.async_copy(table_hbm.at[idx_v.at[j]], rows, sem).wait()

            @pl.loop(0, CHUNK)
            def row_loop(i):
                r = rows.at[i]
                for c in range(D // L):
                    r[pl.ds(c * L, L)] = r[pl.ds(c * L, L)] * SCALE

            pltpu.sync_copy(rows, out_hbm.at[pl.ds(base + j * CHUNK, CHUNK)])

    return k


def kernel(x, table):
    B = x.shape[0] * x.shape[1]
    D = table.shape[1]
    idx3 = x.reshape(NW, B // (NW * CHUNK), CHUNK).astype(jnp.int32)
    out = _make_sc_kernel(B, D)(table, idx3)
    return out.reshape(x.shape[0], x.shape[1], D)


# speedup vs baseline: 1.0987x; 1.0987x over previous
import functools
import math

import jax
import jax.numpy as jnp
from jax import lax
from jax.experimental import pallas as pl
from jax.experimental.pallas import tpu as pltpu
from jax.experimental.pallas import tpu_sc as plsc

D_MODEL = 64
SCALE = math.sqrt(D_MODEL)

NC = 2
NS = 16
NW = NC * NS
L = 16
CHUNK = 128
NBUF = 4


def _make_sc_kernel(B, D):
    assert B % (NW * CHUNK) == 0
    n_chunks = B // (NW * CHUNK)
    assert n_chunks % NBUF == 0 and n_chunks // NBUF >= 2
    groups = n_chunks // NBUF
    per_w = n_chunks * CHUNK
    mesh = plsc.VectorSubcoreMesh(core_axis_name="c", subcore_axis_name="s")

    @functools.partial(
        pl.kernel,
        mesh=mesh,
        out_type=jax.ShapeDtypeStruct((B, D), jnp.float32),
        scratch_types=[
            pltpu.VMEM((n_chunks, CHUNK), jnp.int32),
            pltpu.VMEM((NBUF, CHUNK, D), jnp.float32),
            pltpu.VMEM((NBUF, CHUNK, D), jnp.float32),
            pltpu.SemaphoreType.DMA((NBUF,)),
            pltpu.SemaphoreType.DMA((NBUF,)),
        ],
        compiler_params=pltpu.CompilerParams(use_tc_tiling_on_sc=False),
    )
    def k(table_hbm, idx_hbm, out_hbm, idx_v, ibuf, obuf, isem, osem):
        wid = lax.axis_index("s") * NC + lax.axis_index("c")
        pltpu.sync_copy(idx_hbm.at[wid], idx_v)
        base = wid * per_w

        def fire_in(j, b):
            pltpu.async_copy(table_hbm.at[idx_v.at[j]], ibuf.at[b], isem.at[b])

        def wait_in(j, b):
            pltpu.make_async_copy(
                table_hbm.at[idx_v.at[j]], ibuf.at[b], isem.at[b]
            ).wait()

        def fire_out(j, b):
            pltpu.async_copy(
                obuf.at[b], out_hbm.at[pl.ds(base + j * CHUNK, CHUNK)], osem.at[b]
            )

        def wait_out(j, b):
            pltpu.make_async_copy(
                obuf.at[b], out_hbm.at[pl.ds(base + j * CHUNK, CHUNK)], osem.at[b]
            ).wait()

        def scale(b):
            @pl.loop(0, CHUNK, unroll=8)
            def row_loop(i):
                src = ibuf.at[b].at[i]
                dst = obuf.at[b].at[i]
                for c in range(D // L):
                    dst[pl.ds(c * L, L)] = src[pl.ds(c * L, L)] * SCALE

        for b in range(NBUF):
            fire_in(b, b)

        for b in range(NBUF):
            wait_in(b, b)
            scale(b)
            fire_in(b + NBUF, b)
            fire_out(b, b)

        @pl.loop(1, groups - 1)
        def group_loop(gi):
            for b in range(NBUF):
                j = gi * NBUF + b
                wait_in(j, b)
                scale(b)
                fire_in(j + NBUF, b)
                wait_out(j - NBUF, b)
                fire_out(j, b)

        for b in range(NBUF):
            j = (groups - 1) * NBUF + b
            wait_in(j, b)
            scale(b)
            wait_out(j - NBUF, b)
            fire_out(j, b)

        for b in range(NBUF):
            j = (groups - 1) * NBUF + b
            wait_out(j, b)

    return k


def kernel(x, table):
    B = x.shape[0] * x.shape[1]
    D = table.shape[1]
    idx3 = x.reshape(NW, B // (NW * CHUNK), CHUNK).astype(jnp.int32)
    out = _make_sc_kernel(B, D)(table, idx3)
    return out.reshape(x.shape[0], x.shape[1], D)
